# Initial kernel scaffold; baseline (speedup 1.0000x reference)
import jax
import jax.numpy as jnp
from jax.experimental import pallas as pl


def kernel(h_states, seq_start_end, last_pos, W1, b1, g1, be1, W2, b2, g2, be2):
    raise NotImplementedError("write your pallas kernel here")


# speedup vs baseline: 17.6706x; 17.6706x over previous
import functools

import jax
import jax.numpy as jnp
from jax import lax
from jax.experimental import pallas as pl
from jax.experimental.pallas import tpu as pltpu


def _stage_a(pos_c_ref, pos_r_ref, h_ref, w1_ref, b1_ref,
             y1_ref, s1_ref, q1_ref, dist_s, x_s):
    B, P, H = h_ref.shape
    BP = B * P
    K = x_s.shape[1] // H
    for b in range(B):
        px_c = pos_c_ref[b, :, 0:1]
        py_c = pos_c_ref[b, :, 1:2]
        px_r = pos_r_ref[b, 0:1, :]
        py_r = pos_r_ref[b, 1:2, :]
        dx = px_c - px_r
        dy = py_c - py_r
        dist_s[b * P:(b + 1) * P, :] = jnp.sqrt(dx * dx + dy * dy)

    dall = dist_s[...]
    kidx = lax.broadcasted_iota(jnp.int32, (BP, P), 1).astype(jnp.float32)
    grp = (lax.broadcasted_iota(jnp.int32, (BP, 1), 0) // P) * P
    grp_f = grp.astype(jnp.float32)
    lane = lax.broadcasted_iota(jnp.int32, (BP, BP), 1).astype(jnp.float32)
    hh = h_ref[...].reshape(BP, H)

    for j in range(K):
        dj = dall[:, j:j + 1]
        lt = dall < dj
        tie = (dall == dj) & (kidx < float(j))
        cnt = jnp.sum((lt | tie).astype(jnp.float32), axis=1, keepdims=True)
        oh = (lane == (cnt + grp_f)).astype(jnp.float32)
        x_s[:, j * H:(j + 1) * H] = jnp.dot(oh, hh,
                                            preferred_element_type=jnp.float32)

    y = jnp.dot(x_s[...], w1_ref[...],
                preferred_element_type=jnp.float32) + b1_ref[...]
    y1_ref[...] = y.reshape(B, P, y.shape[-1])

    @pl.when(pl.program_id(0) == 0)
    def _():
        s1_ref[...] = jnp.zeros_like(s1_ref)
        q1_ref[...] = jnp.zeros_like(q1_ref)

    s1_ref[...] += jnp.sum(y, axis=0, keepdims=True)
    q1_ref[...] += jnp.sum(y * y, axis=0, keepdims=True)


def _stage_b(y1_ref, s1_ref, q1_ref, g1_ref, be1_ref, w2_ref, b2_ref,
             y2_ref, s2_ref, q2_ref, *, n):
    m = s1_ref[...] / n
    v = q1_ref[...] / n - m * m
    scale = g1_ref[...] * lax.rsqrt(v + 1e-5)
    a = (y1_ref[...] - m) * scale + be1_ref[...]
    a = jnp.where(a >= 0, a, 0.01 * a)
    y = jnp.dot(a, w2_ref[...], preferred_element_type=jnp.float32) + b2_ref[...]
    y2_ref[...] = y

    @pl.when(pl.program_id(0) == 0)
    def _():
        s2_ref[...] = jnp.zeros_like(s2_ref)
        q2_ref[...] = jnp.zeros_like(q2_ref)

    s2_ref[...] += jnp.sum(y, axis=0, keepdims=True)
    q2_ref[...] += jnp.sum(y * y, axis=0, keepdims=True)


def _stage_c(y2_ref, s2_ref, q2_ref, g2_ref, be2_ref, out_ref, *, n):
    m = s2_ref[...] / n
    v = q2_ref[...] / n - m * m
    scale = g2_ref[...] * lax.rsqrt(v + 1e-5)
    a = (y2_ref[...] - m) * scale + be2_ref[...]
    out_ref[...] = jnp.where(a >= 0, a, 0.01 * a)


def kernel(h_states, seq_start_end, last_pos, W1, b1, g1, be1, W2, b2, g2, be2):
    G = seq_start_end.shape[0]
    N, H = h_states.shape
    P = N // G
    D1 = W1.shape[1]
    D2 = W2.shape[1]
    K = W1.shape[0] // H
    B = 4 if G % 4 == 0 else 1
    NB = G // B
    BP = B * P

    h3 = h_states.reshape(G, P, H)
    pos3 = last_pos.reshape(G, P, 2)
    pos3t = pos3.transpose(0, 2, 1)

    y1, s1, q1 = pl.pallas_call(
        _stage_a,
        grid=(NB,),
        in_specs=[
            pl.BlockSpec((B, P, 2), lambda g: (g, 0, 0)),
            pl.BlockSpec((B, 2, P), lambda g: (g, 0, 0)),
            pl.BlockSpec((B, P, H), lambda g: (g, 0, 0)),
            pl.BlockSpec((K * H, D1), lambda g: (0, 0)),
            pl.BlockSpec((1, D1), lambda g: (0, 0)),
        ],
        out_specs=[
            pl.BlockSpec((B, P, D1), lambda g: (g, 0, 0)),
            pl.BlockSpec((1, D1), lambda g: (0, 0)),
            pl.BlockSpec((1, D1), lambda g: (0, 0)),
        ],
        out_shape=[
            jax.ShapeDtypeStruct((G, P, D1), jnp.float32),
            jax.ShapeDtypeStruct((1, D1), jnp.float32),
            jax.ShapeDtypeStruct((1, D1), jnp.float32),
        ],
        scratch_shapes=[
            pltpu.VMEM((BP, P), jnp.float32),
            pltpu.VMEM((BP, K * H), jnp.float32),
        ],
    )(pos3, pos3t, h3, W1, b1.reshape(1, D1))

    y1f = y1.reshape(N, D1)
    RB = 512

    y2, s2, q2 = pl.pallas_call(
        functools.partial(_stage_b, n=float(N)),
        grid=(N // RB,),
        in_specs=[
            pl.BlockSpec((RB, D1), lambda i: (i, 0)),
            pl.BlockSpec((1, D1), lambda i: (0, 0)),
            pl.BlockSpec((1, D1), lambda i: (0, 0)),
            pl.BlockSpec((1, D1), lambda i: (0, 0)),
            pl.BlockSpec((1, D1), lambda i: (0, 0)),
            pl.BlockSpec((D1, D2), lambda i: (0, 0)),
            pl.BlockSpec((1, D2), lambda i: (0, 0)),
        ],
        out_specs=[
            pl.BlockSpec((RB, D2), lambda i: (i, 0)),
            pl.BlockSpec((1, D2), lambda i: (0, 0)),
            pl.BlockSpec((1, D2), lambda i: (0, 0)),
        ],
        out_shape=[
            jax.ShapeDtypeStruct((N, D2), jnp.float32),
            jax.ShapeDtypeStruct((1, D2), jnp.float32),
            jax.ShapeDtypeStruct((1, D2), jnp.float32),
        ],
    )(y1f, s1, q1, g1.reshape(1, D1), be1.reshape(1, D1), W2,
      b2.reshape(1, D2))

    out = pl.pallas_call(
        functools.partial(_stage_c, n=float(N)),
        grid=(N // RB,),
        in_specs=[
            pl.BlockSpec((RB, D2), lambda i: (i, 0)),
            pl.BlockSpec((1, D2), lambda i: (0, 0)),
            pl.BlockSpec((1, D2), lambda i: (0, 0)),
            pl.BlockSpec((1, D2), lambda i: (0, 0)),
            pl.BlockSpec((1, D2), lambda i: (0, 0)),
        ],
        out_specs=pl.BlockSpec((RB, D2), lambda i: (i, 0)),
        out_shape=jax.ShapeDtypeStruct((N, D2), jnp.float32),
    )(y2, s2, q2, g2.reshape(1, D2), be2.reshape(1, D2))

    return out
